# baseline (speedup 1.0000x reference)
import functools

import jax
import jax.numpy as jnp
from jax import lax
from jax.experimental import pallas as pl
from jax.experimental.pallas import tpu as pltpu
from jax.experimental.pallas import tpu_sc as plsc

N = 10000
E = 320000
C = 8
D = 16
ROW = C * D

NC = 2
NS = 16
NW = NC * NS
EDGES_PER_W = E // NW
CHUNK = 40
NCHUNK = EDGES_PER_W // CHUNK
ZROWS = 8
NZCH = N // ZROWS


def _vfull(c):
    return jnp.full((16,), c, dtype=jnp.float32)


def _rsqrt16(x):
    i = lax.bitcast_convert_type(x, jnp.int32)
    i = jnp.int32(0x5F3759DF) - (i >> 1)
    y = lax.bitcast_convert_type(i, jnp.float32)
    for _ in range(3):
        y = y * (_vfull(1.5) - _vfull(0.5) * x * y * y)
    return y


def _arccos_over_sin(ip):
    a = jnp.abs(ip)
    pa = _vfull(-0.0012624911)
    for c in (0.0066700901, -0.0170881256, 0.0308918810,
              -0.0501743046, 0.0889789874, -0.2145988016, 1.5707963050):
        pa = pa * a + _vfull(c)
    rp = _rsqrt16(_vfull(1.0) + a)
    rm = _rsqrt16(_vfull(1.0) - a)
    neg = ip < _vfull(0.0)
    c1 = jnp.where(neg, _vfull(3.14159265358979) * rm, _vfull(0.0))
    c2 = jnp.where(neg, _vfull(-1.0), _vfull(1.0))
    return (c1 + c2 * pa) * rp


def _sc_edge_body(x2, eidx2, out, A,
                  ib0, ib1, ds0, ds1, q0, q1, p0, p1, l0, l1, zbuf,
                  gsem0, gsem1, isem0, isem1, asem0, asem1):
    cid = lax.axis_index("c")
    sid = lax.axis_index("s")
    wid = sid * NC + cid

    lane = lax.iota(jnp.int32, 16)
    rofs = lane >> 3
    cbase = (lane & 7) * 16

    zv = _vfull(0.0)

    def _zero_row(i, _):
        for j in range(ROW // 16):
            zbuf[i, pl.ds(j * 16, 16)] = zv
        return 0

    lax.fori_loop(0, ZROWS, _zero_row, 0)

    def _zero_chunk(k, _):
        cidx = sid + k * NS

        @pl.when(cidx < NZCH)
        def _():
            pltpu.sync_copy(zbuf, A.at[pl.ds(cidx * ZROWS, ZROWS)])

        return 0

    lax.fori_loop(0, (NZCH + NS - 1) // NS, _zero_chunk, 0)
    plsc.subcore_barrier()

    cbase0 = wid * NCHUNK

    def _compute(qr, pr_, lr):
        @plsc.parallel_loop(0, CHUNK // 2, unroll=4)
        def _pair(pr):
            r = pr * 2 + rofs
            acc = [_vfull(0.0) for _ in range(4)]
            for d in range(D):
                col = cbase + d
                pv = plsc.load_gather(pr_, [r, col])
                qv = plsc.load_gather(qr, [r, col])
                acc[d & 3] = acc[d & 3] + pv * qv
            ip = (acc[0] + acc[1]) + (acc[2] + acc[3])
            ip = jnp.minimum(jnp.maximum(ip, _vfull(-1.0 + 1e-7)),
                             _vfull(1.0 - 1e-7))
            f = _arccos_over_sin(ip)
            g = f * ip
            for d in range(D):
                col = cbase + d
                pv = plsc.load_gather(pr_, [r, col])
                qv = plsc.load_gather(qr, [r, col])
                plsc.store_scatter(lr, [r, col], f * qv - g * pv)

    pltpu.sync_copy(eidx2.at[cbase0], ib0)
    pltpu.async_copy(x2.at[ib0.at[0]], q0, gsem0)
    pltpu.async_copy(x2.at[ib0.at[1]], p0, gsem0)
    pltpu.async_copy(eidx2.at[cbase0 + 1], ib1, isem1)

    def _outer(jj, _):
        for b in range(2):
            j = jj * 2 + b
            ib_p, ds_p, q_p, p_p, l_p = (ib0, ds0, q0, p0, l0) if b == 0 \
                else (ib1, ds1, q1, p1, l1)
            ib_n, q_n, p_n = (ib1, q1, p1) if b == 0 else (ib0, q0, p0)
            gsem_p, isem_p, asem_p = (gsem0, isem0, asem0) if b == 0 \
                else (gsem1, isem1, asem1)
            gsem_n, isem_n = (gsem1, isem1) if b == 0 else (gsem0, isem0)

            @pl.when(j >= 2)
            def _():
                pltpu.make_async_copy(l_p, A.at[ds_p], asem_p).wait()

            @pl.when(j <= NCHUNK - 2)
            def _():
                pltpu.make_async_copy(eidx2.at[0], ib_n, isem_n).wait()
                pltpu.async_copy(x2.at[ib_n.at[0]], q_n, gsem_n)
                pltpu.async_copy(x2.at[ib_n.at[1]], p_n, gsem_n)

            pltpu.make_async_copy(x2.at[pl.ds(0, CHUNK)], q_p, gsem_p).wait()
            pltpu.make_async_copy(x2.at[pl.ds(0, CHUNK)], p_p, gsem_p).wait()
            for off in (0, 16, CHUNK - 16):
                ds_p[pl.ds(off, 16)] = ib_p[1, pl.ds(off, 16)]

            @pl.when(j <= NCHUNK - 3)
            def _():
                pltpu.async_copy(eidx2.at[cbase0 + j + 2], ib_p, isem_p)

            _compute(q_p, p_p, l_p)
            pltpu.async_copy(l_p, A.at[ds_p], asem_p, add=True)
        return 0

    lax.fori_loop(0, NCHUNK // 2, _outer, 0)
    pltpu.make_async_copy(l0, A.at[ds0], asem0).wait()
    pltpu.make_async_copy(l1, A.at[ds1], asem1).wait()
    plsc.subcore_barrier()

    def _dump(k, _):
        cidx = sid + k * NS

        @pl.when(cidx < NZCH)
        def _():
            base = cidx * ZROWS
            pltpu.sync_copy(A.at[pl.ds(base, ZROWS)], zbuf)
            pltpu.sync_copy(zbuf, out.at[cid, pl.ds(base, ZROWS)])

        return 0

    lax.fori_loop(0, (NZCH + NS - 1) // NS, _dump, 0)


@jax.jit
def _sc_edge(x2, eidx2):
    mesh = plsc.VectorSubcoreMesh(core_axis_name="c", subcore_axis_name="s")
    return pl.kernel(
        _sc_edge_body,
        out_type=jax.ShapeDtypeStruct((NC, N, ROW), jnp.float32),
        mesh=mesh,
        compiler_params=pltpu.CompilerParams(needs_layout_passes=False),
        scratch_types=[
            pltpu.VMEM_SHARED((N, ROW), jnp.float32),
            pltpu.VMEM((2, CHUNK), jnp.int32),
            pltpu.VMEM((2, CHUNK), jnp.int32),
            pltpu.VMEM((CHUNK,), jnp.int32),
            pltpu.VMEM((CHUNK,), jnp.int32),
            pltpu.VMEM((CHUNK, ROW), jnp.float32),
            pltpu.VMEM((CHUNK, ROW), jnp.float32),
            pltpu.VMEM((CHUNK, ROW), jnp.float32),
            pltpu.VMEM((CHUNK, ROW), jnp.float32),
            pltpu.VMEM((CHUNK, ROW), jnp.float32),
            pltpu.VMEM((CHUNK, ROW), jnp.float32),
            pltpu.VMEM((ZROWS, ROW), jnp.float32),
            pltpu.SemaphoreType.DMA,
            pltpu.SemaphoreType.DMA,
            pltpu.SemaphoreType.DMA,
            pltpu.SemaphoreType.DMA,
            pltpu.SemaphoreType.DMA,
            pltpu.SemaphoreType.DMA,
        ],
    )(x2, eidx2)


def _tc_node_body(x2, part, tb, db, bm, out):
    X = x2[...]
    Bm = bm[...]
    v = -(part[0] + part[1])
    nrm2 = jnp.dot(v * v, Bm, preferred_element_type=jnp.float32,
                   precision=jax.lax.Precision.HIGHEST)
    nrm = jnp.sqrt(nrm2 + 1e-8)
    act = jax.nn.sigmoid(nrm - db[...])
    gate = (nrm * act >= 0.001).astype(jnp.float32)
    v2 = -(act * gate * tb[...]) * v
    nv2 = jnp.dot(v2 * v2, Bm, preferred_element_type=jnp.float32,
                  precision=jax.lax.Precision.HIGHEST)
    nv = jnp.sqrt(nv2)
    sl = jnp.where(nv < 1e-6, 1.0, jnp.sin(nv) / jnp.maximum(nv, 1e-30))
    o = jnp.cos(nv) * X + sl * v2
    on2 = jnp.dot(o * o, Bm, preferred_element_type=jnp.float32,
                  precision=jax.lax.Precision.HIGHEST)
    out[...] = o / (jnp.sqrt(on2) + 1e-12)


@jax.jit
def _tc_node(x2, part, tb, db, bm):
    nb = 400
    grid = N // nb
    return pl.pallas_call(
        _tc_node_body,
        grid=(grid,),
        in_specs=[
            pl.BlockSpec((nb, ROW), lambda i: (i, 0)),
            pl.BlockSpec((NC, nb, ROW), lambda i: (0, i, 0)),
            pl.BlockSpec((1, ROW), lambda i: (0, 0)),
            pl.BlockSpec((1, ROW), lambda i: (0, 0)),
            pl.BlockSpec((ROW, ROW), lambda i: (0, 0)),
        ],
        out_specs=pl.BlockSpec((nb, ROW), lambda i: (i, 0)),
        out_shape=jax.ShapeDtypeStruct((N, ROW), jnp.float32),
    )(x2, part, tb, db, bm)


def kernel(x, edge_index, t_sqrt, delta_sqrt):
    x2 = x.reshape(N, ROW)
    eidx2 = jnp.stack([edge_index[0].reshape(-1, CHUNK),
                       edge_index[1].reshape(-1, CHUNK)], axis=1)
    part = _sc_edge(x2, eidx2)
    tb = jnp.repeat(t_sqrt.astype(jnp.float32) ** 2, D)[None, :]
    db = jnp.repeat(delta_sqrt.astype(jnp.float32) ** 2, D)[None, :]
    bm = jnp.kron(jnp.eye(C, dtype=jnp.float32),
                  jnp.ones((D, D), dtype=jnp.float32))
    out = _tc_node(x2, part, tb, db, bm)
    return out.reshape(N, C, D)


# speedup vs baseline: 2.3773x; 2.3773x over previous
import functools

import jax
import jax.numpy as jnp
from jax import lax
from jax.experimental import pallas as pl
from jax.experimental.pallas import tpu as pltpu
from jax.experimental.pallas import tpu_sc as plsc

N = 10000
E = 320000
C = 8
D = 16
ROW = C * D

NC = 2
NS = 16
NW = NC * NS
EDGES_PER_W = E // NW
CHUNK = 40
NCHUNK = EDGES_PER_W // CHUNK
ZROWS = 8
NZCH = N // ZROWS


def _vfull(c):
    return jnp.full((16,), c, dtype=jnp.float32)


def _rsqrt16(x):
    i = lax.bitcast_convert_type(x, jnp.int32)
    i = jnp.int32(0x5F3759DF) - (i >> 1)
    y = lax.bitcast_convert_type(i, jnp.float32)
    for _ in range(3):
        y = y * (_vfull(1.5) - _vfull(0.5) * x * y * y)
    return y


def _arccos_over_sin(ip):
    a = jnp.abs(ip)
    pa = _vfull(-0.0012624911)
    for c in (0.0066700901, -0.0170881256, 0.0308918810,
              -0.0501743046, 0.0889789874, -0.2145988016, 1.5707963050):
        pa = pa * a + _vfull(c)
    rp = _rsqrt16(_vfull(1.0) + a)
    rm = _rsqrt16(_vfull(1.0) - a)
    neg = ip < _vfull(0.0)
    c1 = jnp.where(neg, _vfull(3.14159265358979) * rm, _vfull(0.0))
    c2 = jnp.where(neg, _vfull(-1.0), _vfull(1.0))
    return (c1 + c2 * pa) * rp


def _sc_edge_body(x2, eidx2, out, A,
                  ib0, ib1, ds0, ds1, q0, q1, p0, p1, l0, l1, zbuf,
                  gsem0, gsem1, isem0, isem1, asem0, asem1):
    cid = lax.axis_index("c")
    sid = lax.axis_index("s")
    wid = sid * NC + cid

    lane = lax.iota(jnp.int32, 16)
    rofs = lane >> 3
    cbase = (lane & 7) * 16

    zv = _vfull(0.0)

    def _zero_row(i, _):
        for j in range(ROW // 16):
            zbuf[i, pl.ds(j * 16, 16)] = zv
        return 0

    lax.fori_loop(0, ZROWS, _zero_row, 0)

    def _zero_chunk(k, _):
        cidx = sid + k * NS

        @pl.when(cidx < NZCH)
        def _():
            pltpu.sync_copy(zbuf, A.at[pl.ds(cidx * ZROWS, ZROWS)])

        return 0

    lax.fori_loop(0, (NZCH + NS - 1) // NS, _zero_chunk, 0)
    plsc.subcore_barrier()

    cbase0 = wid * NCHUNK

    def _compute(qr, pr_, lr):
        def _pair(pr, _):
            r = pr * 2 + rofs
            acc = [_vfull(0.0) for _ in range(4)]
            pvs = []
            qvs = []
            for d in range(D):
                col = cbase + d
                pv = plsc.load_gather(pr_, [r, col])
                qv = plsc.load_gather(qr, [r, col])
                pvs.append(pv)
                qvs.append(qv)
                acc[d & 3] = acc[d & 3] + pv * qv
            ip = (acc[0] + acc[1]) + (acc[2] + acc[3])
            ip = jnp.minimum(jnp.maximum(ip, _vfull(-1.0 + 1e-7)),
                             _vfull(1.0 - 1e-7))
            f = _arccos_over_sin(ip)
            g = f * ip
            for d in range(D):
                plsc.store_scatter(lr, [r, cbase + d],
                                   f * qvs[d] - g * pvs[d])
            return 0

        lax.fori_loop(0, CHUNK // 2, _pair, 0)

    pltpu.sync_copy(eidx2.at[cbase0], ib0)
    pltpu.async_copy(x2.at[ib0.at[0]], q0, gsem0)
    pltpu.async_copy(x2.at[ib0.at[1]], p0, gsem0)
    pltpu.async_copy(eidx2.at[cbase0 + 1], ib1, isem1)

    def _outer(jj, _):
        for b in range(2):
            j = jj * 2 + b
            ib_p, ds_p, q_p, p_p, l_p = (ib0, ds0, q0, p0, l0) if b == 0 \
                else (ib1, ds1, q1, p1, l1)
            ib_n, q_n, p_n = (ib1, q1, p1) if b == 0 else (ib0, q0, p0)
            gsem_p, isem_p, asem_p = (gsem0, isem0, asem0) if b == 0 \
                else (gsem1, isem1, asem1)
            gsem_n, isem_n = (gsem1, isem1) if b == 0 else (gsem0, isem0)

            @pl.when(j >= 2)
            def _():
                pltpu.make_async_copy(l_p, A.at[ds_p], asem_p).wait()

            @pl.when(j <= NCHUNK - 2)
            def _():
                pltpu.make_async_copy(eidx2.at[0], ib_n, isem_n).wait()
                pltpu.async_copy(x2.at[ib_n.at[0]], q_n, gsem_n)
                pltpu.async_copy(x2.at[ib_n.at[1]], p_n, gsem_n)

            pltpu.make_async_copy(x2.at[pl.ds(0, CHUNK)], q_p, gsem_p).wait()
            pltpu.make_async_copy(x2.at[pl.ds(0, CHUNK)], p_p, gsem_p).wait()
            for off in (0, 16, CHUNK - 16):
                ds_p[pl.ds(off, 16)] = ib_p[1, pl.ds(off, 16)]

            @pl.when(j <= NCHUNK - 3)
            def _():
                pltpu.async_copy(eidx2.at[cbase0 + j + 2], ib_p, isem_p)

            _compute(q_p, p_p, l_p)
            pltpu.async_copy(l_p, A.at[ds_p], asem_p, add=True)
        return 0

    lax.fori_loop(0, NCHUNK // 2, _outer, 0)
    pltpu.make_async_copy(l0, A.at[ds0], asem0).wait()
    pltpu.make_async_copy(l1, A.at[ds1], asem1).wait()
    plsc.subcore_barrier()

    def _dump(k, _):
        cidx = sid + k * NS

        @pl.when(cidx < NZCH)
        def _():
            base = cidx * ZROWS
            pltpu.sync_copy(A.at[pl.ds(base, ZROWS)], zbuf)
            pltpu.sync_copy(zbuf, out.at[cid, pl.ds(base, ZROWS)])

        return 0

    lax.fori_loop(0, (NZCH + NS - 1) // NS, _dump, 0)


@jax.jit
def _sc_edge(x2, eidx2):
    mesh = plsc.VectorSubcoreMesh(core_axis_name="c", subcore_axis_name="s")
    return pl.kernel(
        _sc_edge_body,
        out_type=jax.ShapeDtypeStruct((NC, N, ROW), jnp.float32),
        mesh=mesh,
        compiler_params=pltpu.CompilerParams(needs_layout_passes=False),
        scratch_types=[
            pltpu.VMEM_SHARED((N, ROW), jnp.float32),
            pltpu.VMEM((2, CHUNK), jnp.int32),
            pltpu.VMEM((2, CHUNK), jnp.int32),
            pltpu.VMEM((CHUNK,), jnp.int32),
            pltpu.VMEM((CHUNK,), jnp.int32),
            pltpu.VMEM((CHUNK, ROW), jnp.float32),
            pltpu.VMEM((CHUNK, ROW), jnp.float32),
            pltpu.VMEM((CHUNK, ROW), jnp.float32),
            pltpu.VMEM((CHUNK, ROW), jnp.float32),
            pltpu.VMEM((CHUNK, ROW), jnp.float32),
            pltpu.VMEM((CHUNK, ROW), jnp.float32),
            pltpu.VMEM((ZROWS, ROW), jnp.float32),
            pltpu.SemaphoreType.DMA,
            pltpu.SemaphoreType.DMA,
            pltpu.SemaphoreType.DMA,
            pltpu.SemaphoreType.DMA,
            pltpu.SemaphoreType.DMA,
            pltpu.SemaphoreType.DMA,
        ],
    )(x2, eidx2)


def _tc_node_body(x2, part, tb, db, bm, out):
    X = x2[...]
    Bm = bm[...]
    v = -(part[0] + part[1])
    nrm2 = jnp.dot(v * v, Bm, preferred_element_type=jnp.float32,
                   precision=jax.lax.Precision.HIGHEST)
    nrm = jnp.sqrt(nrm2 + 1e-8)
    act = jax.nn.sigmoid(nrm - db[...])
    gate = (nrm * act >= 0.001).astype(jnp.float32)
    v2 = -(act * gate * tb[...]) * v
    nv2 = jnp.dot(v2 * v2, Bm, preferred_element_type=jnp.float32,
                  precision=jax.lax.Precision.HIGHEST)
    nv = jnp.sqrt(nv2)
    sl = jnp.where(nv < 1e-6, 1.0, jnp.sin(nv) / jnp.maximum(nv, 1e-30))
    o = jnp.cos(nv) * X + sl * v2
    on2 = jnp.dot(o * o, Bm, preferred_element_type=jnp.float32,
                  precision=jax.lax.Precision.HIGHEST)
    out[...] = o / (jnp.sqrt(on2) + 1e-12)


@jax.jit
def _tc_node(x2, part, tb, db, bm):
    nb = 400
    grid = N // nb
    return pl.pallas_call(
        _tc_node_body,
        grid=(grid,),
        in_specs=[
            pl.BlockSpec((nb, ROW), lambda i: (i, 0)),
            pl.BlockSpec((NC, nb, ROW), lambda i: (0, i, 0)),
            pl.BlockSpec((1, ROW), lambda i: (0, 0)),
            pl.BlockSpec((1, ROW), lambda i: (0, 0)),
            pl.BlockSpec((ROW, ROW), lambda i: (0, 0)),
        ],
        out_specs=pl.BlockSpec((nb, ROW), lambda i: (i, 0)),
        out_shape=jax.ShapeDtypeStruct((N, ROW), jnp.float32),
    )(x2, part, tb, db, bm)


def kernel(x, edge_index, t_sqrt, delta_sqrt):
    x2 = x.reshape(N, ROW)
    eidx2 = jnp.stack([edge_index[0].reshape(-1, CHUNK),
                       edge_index[1].reshape(-1, CHUNK)], axis=1)
    part = _sc_edge(x2, eidx2)
    tb = jnp.repeat(t_sqrt.astype(jnp.float32) ** 2, D)[None, :]
    db = jnp.repeat(delta_sqrt.astype(jnp.float32) ** 2, D)[None, :]
    bm = jnp.kron(jnp.eye(C, dtype=jnp.float32),
                  jnp.ones((D, D), dtype=jnp.float32))
    out = _tc_node(x2, part, tb, db, bm)
    return out.reshape(N, C, D)


# speedup vs baseline: 2.3773x; 1.0000x over previous
import jax
import jax.numpy as jnp
from jax import lax
from jax.experimental import pallas as pl
from jax.experimental.pallas import tpu as pltpu
from jax.experimental.pallas import tpu_sc as plsc

N = 10000
E = 320000
C = 8
D = 16
ROW = C * D

NC = 2
NS = 16
NW = NC * NS
EDGES_PER_W = E // NW
CHUNK = 40
NCHUNK = EDGES_PER_W // CHUNK
ZROWS = 8
NZCH = N // ZROWS


def _vfull(c):
    return jnp.full((16,), c, dtype=jnp.float32)


def _rsqrt16(x):
    i = lax.bitcast_convert_type(x, jnp.int32)
    i = jnp.int32(0x5F3759DF) - (i >> 1)
    y = lax.bitcast_convert_type(i, jnp.float32)
    for _ in range(3):
        y = y * (_vfull(1.5) - _vfull(0.5) * x * y * y)
    return y


def _arccos_over_sin(ip):
    a = jnp.abs(ip)
    pa = _vfull(-0.0012624911)
    for c in (0.0066700901, -0.0170881256, 0.0308918810,
              -0.0501743046, 0.0889789874, -0.2145988016, 1.5707963050):
        pa = pa * a + _vfull(c)
    rp = _rsqrt16(_vfull(1.0) + a)
    rm = _rsqrt16(_vfull(1.0) - a)
    neg = ip < _vfull(0.0)
    c1 = jnp.where(neg, _vfull(3.14159265358979) * rm, _vfull(0.0))
    c2 = jnp.where(neg, _vfull(-1.0), _vfull(1.0))
    return (c1 + c2 * pa) * rp


def _sc_edge_body(x2, eidx2, out, A,
                  ib0, ib1, ds0, ds1, q0, q1, p0, p1, l0, l1, zbuf,
                  gsem0, gsem1, isem0, isem1, asem0, asem1):
    cid = lax.axis_index("c")
    sid = lax.axis_index("s")
    wid = sid * NC + cid

    lane = lax.iota(jnp.int32, 16)
    rofs = lane >> 3
    cbase = (lane & 7) * 16

    zv = _vfull(0.0)

    def _zero_row(i, _):
        for j in range(ROW // 16):
            zbuf[i, pl.ds(j * 16, 16)] = zv
        return 0

    lax.fori_loop(0, ZROWS, _zero_row, 0)

    def _zero_chunk(k, _):
        cidx = sid + k * NS

        @pl.when(cidx < NZCH)
        def _():
            pltpu.sync_copy(zbuf, A.at[pl.ds(cidx * ZROWS, ZROWS)])

        return 0

    lax.fori_loop(0, (NZCH + NS - 1) // NS, _zero_chunk, 0)
    plsc.subcore_barrier()

    cbase0 = wid * NCHUNK

    def _compute(qr, pr_, lr):
        def _pair(pr, _):
            r = pr * 2 + rofs
            acc = [_vfull(0.0) for _ in range(4)]
            pvs = []
            qvs = []
            for d in range(D):
                col = cbase + d
                pv = plsc.load_gather(pr_, [r, col])
                qv = plsc.load_gather(qr, [r, col])
                pvs.append(pv)
                qvs.append(qv)
                acc[d & 3] = acc[d & 3] + pv * qv
            ip = (acc[0] + acc[1]) + (acc[2] + acc[3])
            ip = jnp.minimum(jnp.maximum(ip, _vfull(-1.0 + 1e-7)),
                             _vfull(1.0 - 1e-7))
            f = _arccos_over_sin(ip)
            g = f * ip
            for d in range(D):
                plsc.store_scatter(lr, [r, cbase + d],
                                   f * qvs[d] - g * pvs[d])
            return 0

        lax.fori_loop(0, CHUNK // 2, _pair, 0)

    pltpu.sync_copy(eidx2.at[cbase0], ib0)
    pltpu.async_copy(x2.at[ib0.at[0]], q0, gsem0)
    pltpu.async_copy(x2.at[ib0.at[1]], p0, gsem0)
    pltpu.async_copy(eidx2.at[cbase0 + 1], ib1, isem1)

    def _outer(jj, _):
        for b in range(2):
            j = jj * 2 + b
            ib_p, ds_p, q_p, p_p, l_p = (ib0, ds0, q0, p0, l0) if b == 0 \
                else (ib1, ds1, q1, p1, l1)
            ib_n, q_n, p_n = (ib1, q1, p1) if b == 0 else (ib0, q0, p0)
            gsem_p, isem_p, asem_p = (gsem0, isem0, asem0) if b == 0 \
                else (gsem1, isem1, asem1)
            gsem_n, isem_n = (gsem1, isem1) if b == 0 else (gsem0, isem0)

            @pl.when(j >= 2)
            def _():
                pltpu.make_async_copy(l_p, A.at[ds_p], asem_p).wait()

            @pl.when(j <= NCHUNK - 2)
            def _():
                pltpu.make_async_copy(eidx2.at[0], ib_n, isem_n).wait()
                pltpu.async_copy(x2.at[ib_n.at[0]], q_n, gsem_n)
                pltpu.async_copy(x2.at[ib_n.at[1]], p_n, gsem_n)

            pltpu.make_async_copy(x2.at[pl.ds(0, CHUNK)], q_p, gsem_p).wait()
            pltpu.make_async_copy(x2.at[pl.ds(0, CHUNK)], p_p, gsem_p).wait()
            for off in (0, 16, CHUNK - 16):
                ds_p[pl.ds(off, 16)] = ib_p[1, pl.ds(off, 16)]

            @pl.when(j <= NCHUNK - 3)
            def _():
                pltpu.async_copy(eidx2.at[cbase0 + j + 2], ib_p, isem_p)

            _compute(q_p, p_p, l_p)
            pltpu.async_copy(l_p, A.at[ds_p], asem_p, add=True)
        return 0

    lax.fori_loop(0, NCHUNK // 2, _outer, 0)
    pltpu.make_async_copy(l0, A.at[ds0], asem0).wait()
    pltpu.make_async_copy(l1, A.at[ds1], asem1).wait()
    plsc.subcore_barrier()

    def _dump(k, _):
        cidx = sid + k * NS

        @pl.when(cidx < NZCH)
        def _():
            base = cidx * ZROWS
            pltpu.sync_copy(A.at[pl.ds(base, ZROWS)], zbuf)
            pltpu.sync_copy(zbuf, out.at[cid, pl.ds(base, ZROWS)])

        return 0

    lax.fori_loop(0, (NZCH + NS - 1) // NS, _dump, 0)


@jax.jit
def _sc_edge(x2, eidx2):
    mesh = plsc.VectorSubcoreMesh(core_axis_name="c", subcore_axis_name="s")
    return pl.kernel(
        _sc_edge_body,
        out_type=jax.ShapeDtypeStruct((NC, N, ROW), jnp.float32),
        mesh=mesh,
        compiler_params=pltpu.CompilerParams(needs_layout_passes=False),
        scratch_types=[
            pltpu.VMEM_SHARED((N, ROW), jnp.float32),
            pltpu.VMEM((2, CHUNK), jnp.int32),
            pltpu.VMEM((2, CHUNK), jnp.int32),
            pltpu.VMEM((CHUNK,), jnp.int32),
            pltpu.VMEM((CHUNK,), jnp.int32),
            pltpu.VMEM((CHUNK, ROW), jnp.float32),
            pltpu.VMEM((CHUNK, ROW), jnp.float32),
            pltpu.VMEM((CHUNK, ROW), jnp.float32),
            pltpu.VMEM((CHUNK, ROW), jnp.float32),
            pltpu.VMEM((CHUNK, ROW), jnp.float32),
            pltpu.VMEM((CHUNK, ROW), jnp.float32),
            pltpu.VMEM((ZROWS, ROW), jnp.float32),
            pltpu.SemaphoreType.DMA,
            pltpu.SemaphoreType.DMA,
            pltpu.SemaphoreType.DMA,
            pltpu.SemaphoreType.DMA,
            pltpu.SemaphoreType.DMA,
            pltpu.SemaphoreType.DMA,
        ],
    )(x2, eidx2)


def _tc_node_body(x2, part, tb, db, bm, out):
    X = x2[...]
    Bm = bm[...]
    v = -(part[0] + part[1])
    nrm2 = jnp.dot(v * v, Bm, preferred_element_type=jnp.float32,
                   precision=jax.lax.Precision.HIGHEST)
    nrm = jnp.sqrt(nrm2 + 1e-8)
    act = jax.nn.sigmoid(nrm - db[...])
    gate = (nrm * act >= 0.001).astype(jnp.float32)
    v2 = -(act * gate * tb[...]) * v
    nv2 = jnp.dot(v2 * v2, Bm, preferred_element_type=jnp.float32,
                  precision=jax.lax.Precision.HIGHEST)
    nv = jnp.sqrt(nv2)
    sl = jnp.where(nv < 1e-6, 1.0, jnp.sin(nv) / jnp.maximum(nv, 1e-30))
    o = jnp.cos(nv) * X + sl * v2
    on2 = jnp.dot(o * o, Bm, preferred_element_type=jnp.float32,
                  precision=jax.lax.Precision.HIGHEST)
    out[...] = o / (jnp.sqrt(on2) + 1e-12)


@jax.jit
def _tc_node(x2, part, tb, db, bm):
    nb = 400
    grid = N // nb
    return pl.pallas_call(
        _tc_node_body,
        grid=(grid,),
        in_specs=[
            pl.BlockSpec((nb, ROW), lambda i: (i, 0)),
            pl.BlockSpec((NC, nb, ROW), lambda i: (0, i, 0)),
            pl.BlockSpec((1, ROW), lambda i: (0, 0)),
            pl.BlockSpec((1, ROW), lambda i: (0, 0)),
            pl.BlockSpec((ROW, ROW), lambda i: (0, 0)),
        ],
        out_specs=pl.BlockSpec((nb, ROW), lambda i: (i, 0)),
        out_shape=jax.ShapeDtypeStruct((N, ROW), jnp.float32),
    )(x2, part, tb, db, bm)


def kernel(x, edge_index, t_sqrt, delta_sqrt):
    x2 = x.reshape(N, ROW)
    eidx2 = jnp.stack([edge_index[0].reshape(-1, CHUNK),
                       edge_index[1].reshape(-1, CHUNK)], axis=1)
    part = _sc_edge(x2, eidx2)
    tb = jnp.repeat(t_sqrt.astype(jnp.float32) ** 2, D)[None, :]
    db = jnp.repeat(delta_sqrt.astype(jnp.float32) ** 2, D)[None, :]
    bm = jnp.kron(jnp.eye(C, dtype=jnp.float32),
                  jnp.ones((D, D), dtype=jnp.float32))
    out = _tc_node(x2, part, tb, db, bm)
    return out.reshape(N, C, D)
